# baseline (speedup 1.0000x reference)
import jax
import jax.numpy as jnp
from jax.experimental import pallas as pl
from jax.experimental.pallas import tpu as pltpu

_B = 512


def _gumbel_const(shape, dtype):
    u = jax.random.uniform(jax.random.key(42), shape,
                           minval=1e-6, maxval=1.0 - 1e-6, dtype=dtype)
    return -jnp.log(-jnp.log(u))


def _body(g0, g1, g2, g3, ids_ref):
    s = (jnp.sum(g0[...], axis=1, keepdims=True)
         + jnp.sum(g1[...], axis=1, keepdims=True)
         + jnp.sum(g2[...], axis=1, keepdims=True)
         + jnp.sum(g3[...], axis=1, keepdims=True))
    ids_ref[...] = s.astype(jnp.int32)


def kernel(x, temperature, codebook):
    n, d = x.shape
    k = codebook.shape[0]
    g = _gumbel_const((n, k), jnp.float32)
    gs = [jnp.copy(g[:, j * 256:(j + 1) * 256]) for j in range(4)]
    ids2 = pl.pallas_call(
        _body,
        grid=(n // _B,),
        in_specs=[pl.BlockSpec((_B, 256), lambda i: (i, 0)) for _ in range(4)],
        out_specs=pl.BlockSpec((_B, 1), lambda i: (i, 0)),
        out_shape=jax.ShapeDtypeStruct((n, 1), jnp.int32),
        compiler_params=pltpu.CompilerParams(
            dimension_semantics=("parallel",)),
    )(*gs)
    return ids2.astype(jnp.float32), ids2[:, 0]


# speedup vs baseline: 1.0996x; 1.0996x over previous
import jax
import jax.numpy as jnp
from jax.experimental import pallas as pl
from jax.experimental.pallas import tpu as pltpu

_B = 512

_ROT = ((13, 15, 26, 6), (17, 29, 16, 24))
_KS = (0, 42, 0x1BD11BF0)


def _gumbel_tile(base, shape, k):
    row = jax.lax.broadcasted_iota(jnp.uint32, shape, 0)
    col = jax.lax.broadcasted_iota(jnp.uint32, shape, 1)
    cnt = row * jnp.uint32(k) + col + base
    x0 = jnp.zeros(shape, jnp.uint32) + jnp.uint32(_KS[0])
    x1 = cnt + jnp.uint32(_KS[1])
    for group in range(5):
        for r in _ROT[group % 2]:
            x0 = x0 + x1
            x1 = (x1 << jnp.uint32(r)) | (x1 >> jnp.uint32(32 - r))
            x1 = x1 ^ x0
        x0 = x0 + jnp.uint32(_KS[(group + 1) % 3])
        x1 = x1 + jnp.uint32((_KS[(group + 2) % 3] + group + 1) & 0xFFFFFFFF)
    bits = x0 ^ x1
    fb = (bits >> jnp.uint32(9)) | jnp.uint32(0x3F800000)
    fl = jax.lax.bitcast_convert_type(fb, jnp.float32) - jnp.float32(1.0)
    mn = jnp.float32(1e-6)
    mx = jnp.float32(1.0 - 1e-6)
    u = jnp.maximum(mn, fl * (mx - mn) + mn)
    return -jnp.log(-jnp.log(u))


def _vq_body(x_ref, t_ref, cb_ref, emb_ref, ids_ref):
    i = pl.program_id(0)
    xb = x_ref[...]
    cb = cb_ref[...]
    k = cb.shape[0]
    s = jax.lax.dot_general(xb, cb, (((1,), (1,)), ((), ())),
                            preferred_element_type=jnp.float32)
    x2 = jnp.sum(xb * xb, axis=1, keepdims=True)
    c2 = jnp.sum(cb * cb, axis=1)[None, :]
    dist = (x2 + c2) - 2.0 * s
    mn = jnp.min(dist, axis=1, keepdims=True)
    iota = jax.lax.broadcasted_iota(jnp.int32, dist.shape, 1)
    ids_ref[...] = jnp.min(jnp.where(dist == mn, iota, k), axis=1,
                           keepdims=True)
    inv_t = 1.0 / t_ref[0]
    base = (i * _B * k).astype(jnp.uint32)
    g = _gumbel_tile(base, dist.shape, k)
    z = g - dist
    m = jnp.max(z, axis=1, keepdims=True)
    e = jnp.exp((z - m) * inv_t)
    w = e / jnp.sum(e, axis=1, keepdims=True)
    emb_ref[...] = jax.lax.dot_general(w, cb, (((1,), (0,)), ((), ())),
                                       preferred_element_type=jnp.float32)


def kernel(x, temperature, codebook):
    n, d = x.shape
    k = codebook.shape[0]
    t1 = jnp.asarray(temperature, jnp.float32).reshape(1)
    emb, ids2 = pl.pallas_call(
        _vq_body,
        grid=(n // _B,),
        in_specs=[
            pl.BlockSpec((_B, d), lambda i: (i, 0)),
            pl.BlockSpec(memory_space=pltpu.SMEM),
            pl.BlockSpec((k, d), lambda i: (0, 0)),
        ],
        out_specs=[
            pl.BlockSpec((_B, d), lambda i: (i, 0)),
            pl.BlockSpec((_B, 1), lambda i: (i, 0)),
        ],
        out_shape=[
            jax.ShapeDtypeStruct((n, d), jnp.float32),
            jax.ShapeDtypeStruct((n, 1), jnp.int32),
        ],
        compiler_params=pltpu.CompilerParams(
            dimension_semantics=("arbitrary",)),
    )(x, t1, codebook)
    return emb, ids2[:, 0]
